# baseline (speedup 1.0000x reference)
import functools

import jax
import jax.numpy as jnp
from jax import lax
from jax.experimental import pallas as pl
from jax.experimental.pallas import tpu as pltpu
from jax.experimental.pallas import tpu_sc as plsc

N = 10000
FEAT = 512
NG = 64
NCLS = 10
NC = 2
NS = 16
NW = NC * NS
LANE = 16
CH = 80
NCHUNK = 4
TILE_ROWS = CH * NCHUNK
NCHW = FEAT // LANE
NEG_INF = float("-inf")


def _flush(acc, g_cur, regs):
    for c in range(NCHW):
        sl = pl.ds(c * LANE, LANE)
        acc[g_cur, sl] = jnp.maximum(acc[g_cur, sl], regs[c])


def _issue(x_hbm, ids_hbm, base, xbuf, idbuf, xsem, isem):
    pltpu.async_copy(x_hbm.at[pl.ds(base, CH)], xbuf, xsem)
    pltpu.async_copy(ids_hbm.at[pl.ds(base, CH)], idbuf, isem)


def _wait(x_hbm, ids_hbm, xbuf, idbuf, xsem, isem):
    pltpu.make_async_copy(x_hbm.at[pl.ds(0, CH)], xbuf, xsem).wait()
    pltpu.make_async_copy(ids_hbm.at[pl.ds(0, CH)], idbuf, isem).wait()


def _process(xbuf, idbuf, acc, carry):
    neg = jnp.full((LANE,), NEG_INF, jnp.float32)

    def group_body(rb, carry):
        g_cur = carry[0]
        regs = list(carry[1:])
        idvec = idbuf[pl.ds(rb * LANE, LANE)]
        g0 = idvec[0]
        g15 = idvec[LANE - 1]
        r0 = rb * LANE
        uniform = g0 == g15
        flush_cond = jnp.logical_or(g0 != g_cur, jnp.logical_not(uniform))

        @pl.when(flush_cond)
        def _():
            _flush(acc, g_cur, regs)

        regs = [jnp.where(flush_cond, neg, r) for r in regs]

        gmax = []
        for c in range(NCHW):
            sl = pl.ds(c * LANE, LANE)
            m = xbuf[r0, sl]
            for j in range(1, LANE):
                m = jnp.maximum(m, xbuf[r0 + j, sl])
            gmax.append(m)
        regs = [
            jnp.where(uniform, jnp.maximum(r, m), r)
            for r, m in zip(regs, gmax)
        ]

        @pl.when(jnp.logical_not(uniform))
        def _():
            for j in range(LANE):
                g = idvec[j]

                def cb(c4, cc):
                    for u in range(4):
                        sl = pl.ds((c4 * 4 + u) * LANE, LANE)
                        acc[g, sl] = jnp.maximum(acc[g, sl], xbuf[r0 + j, sl])
                    return cc

                lax.fori_loop(0, NCHW // 4, cb, 0)

        return (g15,) + tuple(regs)

    return lax.fori_loop(0, CH // LANE, group_body, carry)


def _seg_max_body(x_hbm, ids_hbm, part_hbm,
                  xbuf0, xbuf1, idbuf0, idbuf1, acc,
                  xsem0, xsem1, isem0, isem1):
    wid = lax.axis_index("c") * NS + lax.axis_index("s")

    def init_g(g, carry):
        for c in range(NCHW):
            acc[g, pl.ds(c * LANE, LANE)] = jnp.full((LANE,), NEG_INF, jnp.float32)
        return carry

    lax.fori_loop(0, NG, init_g, 0)

    base0 = wid * TILE_ROWS

    def cbase(k):
        return jnp.minimum(base0 + k * CH, N - CH)

    _issue(x_hbm, ids_hbm, cbase(0), xbuf0, idbuf0, xsem0, isem0)
    _issue(x_hbm, ids_hbm, cbase(1), xbuf1, idbuf1, xsem1, isem1)

    neg = jnp.full((LANE,), NEG_INF, jnp.float32)
    carry0 = (jnp.int32(0),) + (neg,) * NCHW

    def pair_body(t, carry):
        k = 2 * t
        _wait(x_hbm, ids_hbm, xbuf0, idbuf0, xsem0, isem0)
        carry = _process(xbuf0, idbuf0, acc, carry)

        @pl.when(k + 2 < NCHUNK)
        def _():
            _issue(x_hbm, ids_hbm, cbase(k + 2), xbuf0, idbuf0, xsem0, isem0)

        _wait(x_hbm, ids_hbm, xbuf1, idbuf1, xsem1, isem1)
        carry = _process(xbuf1, idbuf1, acc, carry)

        @pl.when(k + 3 < NCHUNK)
        def _():
            _issue(x_hbm, ids_hbm, cbase(k + 3), xbuf1, idbuf1, xsem1, isem1)

        return carry

    carry = lax.fori_loop(0, NCHUNK // 2, pair_body, carry0)
    _flush(acc, carry[0], list(carry[1:]))
    pltpu.sync_copy(acc, part_hbm.at[wid])


@functools.cache
def _seg_max():
    return functools.partial(
        pl.kernel,
        out_type=jax.ShapeDtypeStruct((NW, NG, FEAT), jnp.float32),
        mesh=plsc.VectorSubcoreMesh(
            core_axis_name="c", subcore_axis_name="s",
            num_cores=NC, num_subcores=NS,
        ),
        scratch_types=[
            pltpu.VMEM((CH, FEAT), jnp.float32),
            pltpu.VMEM((CH, FEAT), jnp.float32),
            pltpu.VMEM((CH,), jnp.int32),
            pltpu.VMEM((CH,), jnp.int32),
            pltpu.VMEM((NG, FEAT), jnp.float32),
            pltpu.SemaphoreType.DMA,
            pltpu.SemaphoreType.DMA,
            pltpu.SemaphoreType.DMA,
            pltpu.SemaphoreType.DMA,
        ],
    )(_seg_max_body)


def _finish_body(part_ref, w_ref, b_ref, out_ref, pooled_ref):
    p = part_ref[0]
    for i in range(1, NW):
        p = jnp.maximum(p, part_ref[i])
    pooled_ref[...] = p
    out_ref[...] = (
        jnp.dot(p, w_ref[...], preferred_element_type=jnp.float32) + b_ref[...]
    )


def kernel(x, edge_index, batch_index, Wl0, Wr0, a0, b0, Wls, Wrs, atts, bs,
           Wout, bout):
    partials = _seg_max()(x, batch_index)
    out, pooled = pl.pallas_call(
        _finish_body,
        out_shape=(
            jax.ShapeDtypeStruct((NG, NCLS), jnp.float32),
            jax.ShapeDtypeStruct((NG, FEAT), jnp.float32),
        ),
    )(partials, Wout, bout.reshape(1, NCLS))
    return (out, pooled)


# speedup vs baseline: 2.0215x; 2.0215x over previous
import functools

import jax
import jax.numpy as jnp
from jax import lax
from jax.experimental import pallas as pl
from jax.experimental.pallas import tpu as pltpu
from jax.experimental.pallas import tpu_sc as plsc

N = 10000
FEAT = 512
NG = 64
NCLS = 10
NEG_INF = float("-inf")

NC = 2
NS = 16
NW = NC * NS
LANE = 16
CH = 64
NCHUNK = 2
TILE_ROWS = CH * NCHUNK
SC_ROWS = NW * TILE_ROWS
NCHW = FEAT // LANE

TCB = 512
TC_FIRST_BLK = 7
TC_BLOCKS = 13


def _flush(acc, g_cur, regs):
    for c in range(NCHW):
        sl = pl.ds(c * LANE, LANE)
        acc[g_cur, sl] = jnp.maximum(acc[g_cur, sl], regs[c])


def _seg_max_body(x_hbm, ids_hbm, part_hbm, xbuf, ids_v, acc):
    wid = lax.axis_index("c") * NS + lax.axis_index("s")

    def init_g(g, carry):
        for c in range(NCHW):
            acc[g, pl.ds(c * LANE, LANE)] = jnp.full((LANE,), NEG_INF, jnp.float32)
        return carry

    lax.fori_loop(0, NG, init_g, 0)

    base0 = wid * TILE_ROWS
    neg = jnp.full((LANE,), NEG_INF, jnp.float32)
    carry0 = (jnp.int32(0),) + (neg,) * NCHW

    def chunk_body(k, carry):
        base = jnp.minimum(base0 + k * CH, N - CH)
        pltpu.sync_copy(x_hbm.at[pl.ds(base, CH)], xbuf)
        pltpu.sync_copy(ids_hbm.at[pl.ds(base, CH)], ids_v)

        def group_body(rb, carry):
            g_cur = carry[0]
            regs = list(carry[1:])
            idvec = ids_v[pl.ds(rb * LANE, LANE)]
            for j in range(LANE):
                g = idvec[j]
                changed = g != g_cur

                @pl.when(changed)
                def _():
                    _flush(acc, g_cur, regs)

                r = rb * LANE + j
                for c in range(NCHW):
                    row_c = xbuf[r, pl.ds(c * LANE, LANE)]
                    regs[c] = jnp.where(changed, row_c, jnp.maximum(regs[c], row_c))
                g_cur = g
            return (g_cur,) + tuple(regs)

        return lax.fori_loop(0, CH // LANE, group_body, carry)

    carry = lax.fori_loop(0, NCHUNK, chunk_body, carry0)
    _flush(acc, carry[0], list(carry[1:]))
    pltpu.sync_copy(acc, part_hbm.at[wid])


@functools.cache
def _seg_max():
    return functools.partial(
        pl.kernel,
        out_type=jax.ShapeDtypeStruct((NW, NG, FEAT), jnp.float32),
        mesh=plsc.VectorSubcoreMesh(
            core_axis_name="c", subcore_axis_name="s",
            num_cores=NC, num_subcores=NS,
        ),
        scratch_types=[
            pltpu.VMEM((CH, FEAT), jnp.float32),
            pltpu.VMEM((CH,), jnp.int32),
            pltpu.VMEM((NG, FEAT), jnp.float32),
        ],
    )(_seg_max_body)


def _tc_seg_body(sp_ref, x_ref, ids_ref, out_ref):
    i = pl.program_id(0)

    @pl.when(i == 0)
    def _():
        out_ref[...] = jnp.full((NG, FEAT), NEG_INF, jnp.float32)

    base = (TC_FIRST_BLK + i) * TCB
    lo = sp_ref[base]
    hi = sp_ref[jnp.minimum(base + TCB - 1, N - 1)]
    rows = base + lax.broadcasted_iota(jnp.int32, (TCB, 1), 0)
    valid = rows < N
    ids = ids_ref[...]
    xblk = x_ref[...]

    def seg_body(s, carry):
        mask = jnp.logical_and(valid, ids == s)
        m = jnp.max(jnp.where(mask, xblk, NEG_INF), axis=0, keepdims=True)
        out_ref[pl.ds(s, 1), :] = jnp.maximum(out_ref[pl.ds(s, 1), :], m)
        return carry

    lax.fori_loop(lo, hi + 1, seg_body, 0)


def _tc_seg_max(x, ids2d, batch_index):
    grid_spec = pltpu.PrefetchScalarGridSpec(
        num_scalar_prefetch=1,
        grid=(TC_BLOCKS,),
        in_specs=[
            pl.BlockSpec((TCB, FEAT), lambda i, sp: (TC_FIRST_BLK + i, 0)),
            pl.BlockSpec((TCB, 1), lambda i, sp: (TC_FIRST_BLK + i, 0)),
        ],
        out_specs=pl.BlockSpec((NG, FEAT), lambda i, sp: (0, 0)),
    )
    return pl.pallas_call(
        _tc_seg_body,
        grid_spec=grid_spec,
        out_shape=jax.ShapeDtypeStruct((NG, FEAT), jnp.float32),
    )(batch_index, x, ids2d)


def _finish_body(part_ref, tc_ref, w_ref, b_ref, out_ref, pooled_ref):
    p = tc_ref[...]
    for i in range(NW):
        p = jnp.maximum(p, part_ref[i])
    pooled_ref[...] = p
    out_ref[...] = (
        jnp.dot(p, w_ref[...], preferred_element_type=jnp.float32) + b_ref[...]
    )


def kernel(x, edge_index, batch_index, Wl0, Wr0, a0, b0, Wls, Wrs, atts, bs,
           Wout, bout):
    partials = _seg_max()(x, batch_index)
    tc_part = _tc_seg_max(x, batch_index.reshape(N, 1), batch_index)
    out, pooled = pl.pallas_call(
        _finish_body,
        out_shape=(
            jax.ShapeDtypeStruct((NG, NCLS), jnp.float32),
            jax.ShapeDtypeStruct((NG, FEAT), jnp.float32),
        ),
    )(partials, tc_part, Wout, bout.reshape(1, NCLS))
    return (out, pooled)
